# baseline (speedup 1.0000x reference)
import functools
import math

import jax
import jax.numpy as jnp
from jax import lax
from jax.experimental import pallas as pl
from jax.experimental.pallas import tpu as pltpu
from jax.experimental.pallas import tpu_sc as plsc

_LANES = 16


@functools.partial(jax.jit, static_argnames=("n_rows", "d_model", "scale"))
def _gather_scale(idx3d, table, n_rows, d_model, scale):
    nw, nch, chb = idx3d.shape
    rows_per_w = nch * chb
    info = plsc.get_sparse_core_info()
    nc = info.num_cores
    mesh = plsc.VectorSubcoreMesh(core_axis_name="c", subcore_axis_name="s")

    @functools.partial(
        pl.kernel,
        mesh=mesh,
        out_type=jax.ShapeDtypeStruct((n_rows, d_model), jnp.float32),
        scratch_types=[
            pltpu.VMEM((nch, chb), jnp.int32),
            pltpu.VMEM((chb, d_model), jnp.float32),
            pltpu.SemaphoreType.DMA,
        ],
        compiler_params=pltpu.CompilerParams(use_tc_tiling_on_sc=False),
    )
    def k(idx_hbm, table_hbm, out_hbm, idx_v, rows_v, sem):
        wid = lax.axis_index("s") * nc + lax.axis_index("c")
        base = wid * rows_per_w
        pltpu.sync_copy(idx_hbm.at[wid], idx_v)

        def chunk(j, carry):
            pltpu.async_copy(table_hbm.at[idx_v.at[j]], rows_v, sem).wait()

            def srow(r, c2):
                for d in range(d_model // _LANES):
                    s = pl.ds(d * _LANES, _LANES)
                    rows_v[r, s] = rows_v[r, s] * scale
                return c2

            lax.fori_loop(0, chb, srow, 0, unroll=8)
            pltpu.sync_copy(rows_v, out_hbm.at[pl.ds(base + j * chb, chb)])
            return carry

        lax.fori_loop(0, nch, chunk, 0)

    return k(idx3d, table)


def kernel(x, table):
    d_model = table.shape[1]
    n_rows = x.size
    scale = math.sqrt(d_model)
    info = plsc.get_sparse_core_info()
    nw = info.num_cores * info.num_subcores
    chb = 128
    nch = n_rows // (nw * chb)
    idx3d = x.reshape(-1).astype(jnp.int32).reshape(nw, nch, chb)
    out = _gather_scale(idx3d, table, n_rows, d_model, scale)
    return out.reshape(x.shape + (d_model,))


# speedup vs baseline: 1.0506x; 1.0506x over previous
import functools
import math

import jax
import jax.numpy as jnp
from jax import lax
from jax.experimental import pallas as pl
from jax.experimental.pallas import tpu as pltpu
from jax.experimental.pallas import tpu_sc as plsc

_LANES = 16
_NBUF = 4
_NOUT = 2


@functools.partial(jax.jit, static_argnames=("n_rows", "d_model", "scale"))
def _gather_scale(idx3d, table, n_rows, d_model, scale):
    nw, nch, chb = idx3d.shape
    rows_per_w = nch * chb
    info = plsc.get_sparse_core_info()
    nc = info.num_cores
    mesh = plsc.VectorSubcoreMesh(core_axis_name="c", subcore_axis_name="s")
    n_groups = nch // _NBUF
    assert nch % _NBUF == 0 and n_groups >= 3

    @functools.partial(
        pl.kernel,
        mesh=mesh,
        out_type=jax.ShapeDtypeStruct((n_rows, d_model), jnp.float32),
        scratch_types=[
            pltpu.VMEM((nch, chb), jnp.int32),
            pltpu.VMEM((_NBUF, chb, d_model), jnp.float32),
            pltpu.VMEM((_NOUT, chb, d_model), jnp.float32),
            pltpu.SemaphoreType.DMA((_NBUF,)),
            pltpu.SemaphoreType.DMA((_NOUT,)),
        ],
        compiler_params=pltpu.CompilerParams(use_tc_tiling_on_sc=False),
    )
    def k(idx_hbm, table_hbm, out_hbm, idx_v, rin, rout, g_sem, o_sem):
        wid = lax.axis_index("s") * nc + lax.axis_index("c")
        base = wid * rows_per_w
        pltpu.sync_copy(idx_hbm.at[wid], idx_v)

        def start_gather(j, b):
            pltpu.async_copy(table_hbm.at[idx_v.at[j]], rin.at[b], g_sem.at[b])

        def wait_gather(j, b):
            pltpu.make_async_copy(
                table_hbm.at[idx_v.at[j]], rin.at[b], g_sem.at[b]
            ).wait()

        def scale_chunk(b, p):
            def srow(r, c2):
                for d in range(d_model // _LANES):
                    s = pl.ds(d * _LANES, _LANES)
                    rout[p, r, s] = rin[b, r, s] * scale
                return c2

            lax.fori_loop(0, chb, srow, 0, unroll=8)

        def start_out(j, p):
            pltpu.async_copy(
                rout.at[p], out_hbm.at[pl.ds(base + j * chb, chb)], o_sem.at[p]
            )

        def wait_out(p):
            pltpu.make_async_copy(
                rout.at[p], out_hbm.at[pl.ds(base, chb)], o_sem.at[p]
            ).wait()

        for b in range(_NBUF):
            start_gather(b, b)

        for b in range(_NBUF):
            j = b
            wait_gather(j, b)
            if j >= _NOUT:
                wait_out(j % _NOUT)
            scale_chunk(b, j % _NOUT)
            start_out(j, j % _NOUT)
            start_gather(j + _NBUF, b)

        def group(g, carry):
            for b in range(_NBUF):
                j = g * _NBUF + b
                wait_gather(j, b)
                wait_out(b % _NOUT)
                scale_chunk(b, b % _NOUT)
                start_out(j, b % _NOUT)
                start_gather(j + _NBUF, b)
            return carry

        lax.fori_loop(1, n_groups - 1, group, 0)

        for b in range(_NBUF):
            j = (n_groups - 1) * _NBUF + b
            wait_gather(j, b)
            wait_out(b % _NOUT)
            scale_chunk(b, b % _NOUT)
            start_out(j, b % _NOUT)

        for p in range(_NOUT):
            wait_out(p)

    return k(idx3d, table)


def kernel(x, table):
    d_model = table.shape[1]
    n_rows = x.size
    scale = math.sqrt(d_model)
    info = plsc.get_sparse_core_info()
    nw = info.num_cores * info.num_subcores
    chb = 128
    nch = n_rows // (nw * chb)
    idx3d = x.reshape(-1).astype(jnp.int32).reshape(nw, nch, chb)
    out = _gather_scale(idx3d, table, n_rows, d_model, scale)
    return out.reshape(x.shape + (d_model,))


# speedup vs baseline: 1.0536x; 1.0029x over previous
import functools
import math

import jax
import jax.numpy as jnp
from jax import lax
from jax.experimental import pallas as pl
from jax.experimental.pallas import tpu as pltpu
from jax.experimental.pallas import tpu_sc as plsc

_LANES = 16
_NG = 4
_NO = 2


@functools.partial(jax.jit, static_argnames=("d_model", "scale"))
def _gather_scale(idx2d, table, d_model, scale):
    nw, rows_per_w = idx2d.shape
    info = plsc.get_sparse_core_info()
    nc = info.num_cores
    mesh = plsc.VectorSubcoreMesh(core_axis_name="c", subcore_axis_name="s")
    seq = 200
    rb_per_w = rows_per_w // seq
    n_batch = nw * rb_per_w
    nch = rb_per_w
    assert nch % _NG == 0 and nch // _NG >= 3

    @functools.partial(
        pl.kernel,
        mesh=mesh,
        out_type=jax.ShapeDtypeStruct((n_batch, seq, d_model), jnp.float32),
        scratch_types=[
            pltpu.VMEM((rows_per_w,), jnp.int32),
            pltpu.VMEM((_NG, seq, d_model), jnp.float32),
            pltpu.VMEM((_NO, 1, seq, d_model), jnp.float32),
            pltpu.SemaphoreType.DMA((_NG,)),
            pltpu.SemaphoreType.DMA((_NO,)),
        ],
        compiler_params=pltpu.CompilerParams(use_tc_tiling_on_sc=False),
    )
    def k(idx_hbm, table_hbm, out_hbm, idx_v, g, o, g_sem, o_sem):
        wid = lax.axis_index("s") * nc + lax.axis_index("c")
        row0 = wid * rb_per_w
        pltpu.sync_copy(idx_hbm.at[wid], idx_v)

        def start_gather(j, b):
            pltpu.async_copy(
                table_hbm.at[idx_v.at[pl.ds(j * seq, seq)]], g.at[b], g_sem.at[b]
            )

        def wait_gather(j, b):
            pltpu.make_async_copy(
                table_hbm.at[idx_v.at[pl.ds(j * seq, seq)]], g.at[b], g_sem.at[b]
            ).wait()

        def scale_chunk(b, p):
            def srow(t, c2):
                for d in range(d_model // _LANES):
                    s = pl.ds(d * _LANES, _LANES)
                    o[p, 0, t, s] = g[b, t, s] * scale
                return c2

            lax.fori_loop(0, seq, srow, 0, unroll=8)

        def start_out(j, p):
            pltpu.async_copy(
                o.at[p], out_hbm.at[pl.ds(row0 + j, 1)], o_sem.at[p]
            )

        def wait_out(p):
            pltpu.make_async_copy(
                o.at[p], out_hbm.at[pl.ds(row0, 1)], o_sem.at[p]
            ).wait()

        for b in range(_NG):
            start_gather(b, b)

        for b in range(_NG):
            wait_gather(b, b)
            if b >= _NO:
                wait_out(b % _NO)
            scale_chunk(b, b % _NO)
            start_out(b, b % _NO)
            start_gather(b + _NG, b)

        n_groups = nch // _NG

        def group(gr, carry):
            for b in range(_NG):
                j = gr * _NG + b
                wait_gather(j, b)
                wait_out(b % _NO)
                scale_chunk(b, b % _NO)
                start_out(j, b % _NO)
                start_gather(j + _NG, b)
            return carry

        lax.fori_loop(1, n_groups - 1, group, 0)

        for b in range(_NG):
            j = (n_groups - 1) * _NG + b
            wait_gather(j, b)
            wait_out(b % _NO)
            scale_chunk(b, b % _NO)
            start_out(j, b % _NO)

        for p in range(_NO):
            wait_out(p)

    return k(idx2d, table)


def kernel(x, table):
    d_model = table.shape[1]
    n_rows = x.size
    scale = math.sqrt(d_model)
    info = plsc.get_sparse_core_info()
    nw = info.num_cores * info.num_subcores
    idx2d = x.reshape(-1).astype(jnp.int32).reshape(nw, n_rows // nw)
    return _gather_scale(idx2d, table, d_model, scale)


# speedup vs baseline: 1.3620x; 1.2927x over previous
import functools
import math

import jax
import jax.numpy as jnp
from jax import lax
from jax.experimental import pallas as pl
from jax.experimental.pallas import tpu as pltpu
from jax.experimental.pallas import tpu_sc as plsc

_LANES = 16
_NG = 4
_NO = 2


@functools.partial(jax.jit, static_argnames=("d_model", "scale"))
def _gather_scale(idx2d, table, d_model, scale):
    nw, rows_per_w = idx2d.shape
    info = plsc.get_sparse_core_info()
    nc = info.num_cores
    mesh = plsc.VectorSubcoreMesh(core_axis_name="c", subcore_axis_name="s")
    seq = 200
    rb_per_w = rows_per_w // seq
    n_batch = nw * rb_per_w
    nch = rb_per_w
    assert nch % _NG == 0 and nch // _NG >= 3

    @functools.partial(
        pl.kernel,
        mesh=mesh,
        out_type=jax.ShapeDtypeStruct((n_batch, seq, 2 * d_model), jnp.float32),
        scratch_types=[
            pltpu.VMEM((rows_per_w,), jnp.int32),
            pltpu.VMEM((_NG, seq, d_model), jnp.float32),
            pltpu.VMEM((_NO, 1, seq, d_model), jnp.float32),
            pltpu.SemaphoreType.DMA((_NG,)),
            pltpu.SemaphoreType.DMA((_NO,)),
        ],
        compiler_params=pltpu.CompilerParams(use_tc_tiling_on_sc=False),
    )
    def k(idx_hbm, table_hbm, out_hbm, idx_v, g, o, g_sem, o_sem):
        wid = lax.axis_index("s") * nc + lax.axis_index("c")
        row0 = wid * rb_per_w
        pltpu.sync_copy(idx_hbm.at[wid], idx_v)

        def start_gather(j, b):
            pltpu.async_copy(
                table_hbm.at[idx_v.at[pl.ds(j * seq, seq)]], g.at[b], g_sem.at[b]
            )

        def wait_gather(j, b):
            pltpu.make_async_copy(
                table_hbm.at[idx_v.at[pl.ds(j * seq, seq)]], g.at[b], g_sem.at[b]
            ).wait()

        def scale_chunk(b, p):
            def srow(t, c2):
                for d in range(d_model // _LANES):
                    s = pl.ds(d * _LANES, _LANES)
                    o[p, 0, t, s] = g[b, t, s] * scale
                return c2

            lax.fori_loop(0, seq, srow, 0, unroll=8)

        def start_out(j, p):
            pltpu.async_copy(
                o.at[p],
                out_hbm.at[pl.ds(row0 + j, 1), slice(None), pl.ds(0, d_model)],
                o_sem.at[p],
            )

        def wait_out(p):
            pltpu.make_async_copy(
                o.at[p],
                out_hbm.at[pl.ds(row0, 1), slice(None), pl.ds(0, d_model)],
                o_sem.at[p],
            ).wait()

        for b in range(_NG):
            start_gather(b, b)

        for b in range(_NG):
            wait_gather(b, b)
            if b >= _NO:
                wait_out(b % _NO)
            scale_chunk(b, b % _NO)
            start_out(b, b % _NO)
            start_gather(b + _NG, b)

        n_groups = nch // _NG

        def group(gr, carry):
            for b in range(_NG):
                j = gr * _NG + b
                wait_gather(j, b)
                wait_out(b % _NO)
                scale_chunk(b, b % _NO)
                start_out(j, b % _NO)
                start_gather(j + _NG, b)
            return carry

        lax.fori_loop(1, n_groups - 1, group, 0)

        for b in range(_NG):
            j = (n_groups - 1) * _NG + b
            wait_gather(j, b)
            wait_out(b % _NO)
            scale_chunk(b, b % _NO)
            start_out(j, b % _NO)

        for p in range(_NO):
            wait_out(p)

    return k(idx2d, table)


def kernel(x, table):
    d_model = table.shape[1]
    n_rows = x.size
    scale = math.sqrt(d_model)
    info = plsc.get_sparse_core_info()
    nw = info.num_cores * info.num_subcores
    idx2d = x.reshape(-1).astype(jnp.int32).reshape(nw, n_rows // nw)
    out_pad = _gather_scale(idx2d, table, d_model, scale)
    return out_pad[:, :, :d_model]
